# baseline (speedup 1.0000x reference)
import jax
import jax.numpy as jnp
from jax.experimental import pallas as pl

_K = 8
_SUB = 4
_BR = 256
_BRANK = 512
_BM = 256


def _knn_kernel(xr_ref, xa_ref, x2r_ref, x2c_ref, drow_ref, dcol_ref):
    xr = xr_ref[0]
    xa = xa_ref[0]
    n = xa.shape[0]
    dotv = jax.lax.dot_general(xr, xa, (((1,), (1,)), ((), ())),
                               preferred_element_type=jnp.float32)
    x2r = x2r_ref[0]
    x2c = x2c_ref[0]
    d2 = (x2r + x2c) - 2.0 * dotv
    d2 = jnp.maximum(d2, 0.0)
    cols = jax.lax.broadcasted_iota(jnp.int32, d2.shape, 1)
    vals = []
    for _ in range(_K):
        m = jnp.min(d2, axis=1, keepdims=True)
        eq = d2 == m
        mi = jnp.min(jnp.where(eq, cols, n), axis=1, keepdims=True)
        d2 = jnp.where(cols == mi, jnp.inf, d2)
        vals.append(jnp.sqrt(m))
    s1 = [vals[i] + vals[i + 4] for i in range(4)]
    s2 = [s1[0] + s1[2], s1[1] + s1[3]]
    dens = (s2[0] + s2[1]) * 0.125
    drow_ref[0, 0, :] = dens[:, 0]
    dcol_ref[0] = dens


def _rank_kernel(dcol_ref, drow_ref, rrow_ref):
    i = pl.program_id(1)
    dn = dcol_ref[0]
    dm = drow_ref[0]
    br, n = dn.shape[0], dm.shape[1]
    cols = jax.lax.broadcasted_iota(jnp.int32, (br, n), 1)
    rows = jax.lax.broadcasted_iota(jnp.int32, (br, n), 0) + i * br
    beats = (dm > dn) | ((dm == dn) & (cols < rows))
    rank = jnp.sum(beats.astype(jnp.int32), axis=1)
    rrow_ref[0, 0, :] = rank


def _gather_kernel(rrow_ref, xa_ref, pos_ref, cam_ref,
                   f_ref, p_ref, c_ref):
    j = pl.program_id(1)
    rank = rrow_ref[0]
    bm = f_ref.shape[1]
    n = rank.shape[1]
    r = jax.lax.broadcasted_iota(jnp.int32, (bm, n), 0) + j * bm
    eq = rank == r
    oneh = eq.astype(jnp.float32)
    f_ref[0] = jax.lax.dot_general(oneh, xa_ref[0], (((1,), (0,)), ((), ())),
                                   precision=jax.lax.Precision.HIGHEST,
                                   preferred_element_type=jnp.float32)
    p_ref[0] = jax.lax.dot_general(oneh, pos_ref[0], (((1,), (0,)), ((), ())),
                                   precision=jax.lax.Precision.HIGHEST,
                                   preferred_element_type=jnp.float32)
    cam = cam_ref[0]
    c_ref[0] = jnp.sum(jnp.where(eq, cam, 0), axis=1, keepdims=True)


def kernel(features, pos, cam_ids):
    b, n, c = features.shape
    m = n // _SUB
    x2 = jnp.sum(features * features, axis=-1)
    x2col = x2.reshape(b, n, 1)
    x2row = x2.reshape(b, 1, n)

    drow, dcol = pl.pallas_call(
        _knn_kernel,
        grid=(b, n // _BR),
        in_specs=[pl.BlockSpec((1, _BR, c), lambda bb, i: (bb, i, 0)),
                  pl.BlockSpec((1, n, c), lambda bb, i: (bb, 0, 0)),
                  pl.BlockSpec((1, _BR, 1), lambda bb, i: (bb, i, 0)),
                  pl.BlockSpec((1, 1, n), lambda bb, i: (bb, 0, 0))],
        out_specs=[pl.BlockSpec((1, 1, _BR), lambda bb, i: (bb, 0, i)),
                   pl.BlockSpec((1, _BR, 1), lambda bb, i: (bb, i, 0))],
        out_shape=[jax.ShapeDtypeStruct((b, 1, n), jnp.float32),
                   jax.ShapeDtypeStruct((b, n, 1), jnp.float32)],
    )(features, features, x2col, x2row)

    rankrow = pl.pallas_call(
        _rank_kernel,
        grid=(b, n // _BRANK),
        in_specs=[pl.BlockSpec((1, _BRANK, 1), lambda bb, i: (bb, i, 0)),
                  pl.BlockSpec((1, 1, n), lambda bb, i: (bb, 0, 0))],
        out_specs=pl.BlockSpec((1, 1, _BRANK), lambda bb, i: (bb, 0, i)),
        out_shape=jax.ShapeDtypeStruct((b, 1, n), jnp.int32),
    )(dcol, drow)

    feats, posg, cam = pl.pallas_call(
        _gather_kernel,
        grid=(b, m // _BM),
        in_specs=[pl.BlockSpec((1, 1, n), lambda bb, j: (bb, 0, 0)),
                  pl.BlockSpec((1, n, c), lambda bb, j: (bb, 0, 0)),
                  pl.BlockSpec((1, n, 3), lambda bb, j: (bb, 0, 0)),
                  pl.BlockSpec((1, 1, n), lambda bb, j: (bb, 0, 0))],
        out_specs=[pl.BlockSpec((1, _BM, c), lambda bb, j: (bb, j, 0)),
                   pl.BlockSpec((1, _BM, 3), lambda bb, j: (bb, j, 0)),
                   pl.BlockSpec((1, _BM, 1), lambda bb, j: (bb, j, 0))],
        out_shape=[jax.ShapeDtypeStruct((b, m, c), jnp.float32),
                   jax.ShapeDtypeStruct((b, m, 3), jnp.float32),
                   jax.ShapeDtypeStruct((b, m, 1), jnp.int32)],
    )(rankrow, features, pos, cam_ids.reshape(b, 1, n))

    return feats, posg, cam.reshape(b, m)


# speedup vs baseline: 1.0945x; 1.0945x over previous
import functools

import jax
import jax.numpy as jnp
from jax import lax
from jax.experimental import pallas as pl
from jax.experimental.pallas import tpu as pltpu
from jax.experimental.pallas import tpu_sc as plsc

_K = 8
_SUB = 4
_BR = 256
_BRANK = 512
_BM = 256
_DS = 128


def _knn_kernel(xr_ref, xa_ref, x2r_ref, x2c_ref, drow_ref, dcol_ref):
    xr = xr_ref[0]
    xa = xa_ref[0]
    n = xa.shape[0]
    dotv = jax.lax.dot_general(xr, xa, (((1,), (1,)), ((), ())),
                               preferred_element_type=jnp.float32)
    x2r = x2r_ref[0]
    x2c = x2c_ref[0]
    d2 = (x2r + x2c) - 2.0 * dotv
    d2 = jnp.maximum(d2, 0.0)
    cols = jax.lax.broadcasted_iota(jnp.int32, d2.shape, 1)
    vals = []
    for _ in range(_K):
        m = jnp.min(d2, axis=1, keepdims=True)
        eq = d2 == m
        mi = jnp.min(jnp.where(eq, cols, n), axis=1, keepdims=True)
        d2 = jnp.where(cols == mi, jnp.inf, d2)
        vals.append(jnp.sqrt(m))
    s1 = [vals[i] + vals[i + 4] for i in range(4)]
    s2 = [s1[0] + s1[2], s1[1] + s1[3]]
    dens = (s2[0] + s2[1]) * 0.125
    drow_ref[0, 0, :] = dens[:, 0]
    dcol_ref[0] = dens


def _rank_kernel(dcol_ref, drow_ref, rrow_ref):
    i = pl.program_id(1)
    dn = dcol_ref[0]
    dm = drow_ref[0]
    br, n = dn.shape[0], dm.shape[1]
    cols = jax.lax.broadcasted_iota(jnp.int32, (br, n), 1)
    rows = jax.lax.broadcasted_iota(jnp.int32, (br, n), 0) + i * br
    beats = (dm > dn) | ((dm == dn) & (cols < rows))
    rank = jnp.sum(beats.astype(jnp.int32), axis=1)
    rrow_ref[0, 0, :] = rank


def _inds_kernel(rrow_ref, i_ref):
    bb = pl.program_id(0)
    j = pl.program_id(1)
    rank = rrow_ref[0]
    bm = i_ref.shape[1]
    n = rank.shape[1]
    r = jax.lax.broadcasted_iota(jnp.int32, (bm, n), 0) + j * bm
    eq = rank == r
    cols = jax.lax.broadcasted_iota(jnp.int32, (bm, n), 1)
    inds = jnp.sum(jnp.where(eq, cols, 0), axis=1, keepdims=True)
    i_ref[0] = inds + bb * n


def _sc_gather_body(nw, bpw, feat_ref, side_ref, idx_ref,
                    of_ref, os_ref, idx_v, rowsf_v, rowss_v, sem):
    wid = lax.axis_index("s") * 2 + lax.axis_index("c")
    base = wid * bpw
    pltpu.sync_copy(idx_ref.at[pl.ds(base, bpw)], idx_v)
    pltpu.async_copy(feat_ref.at[idx_v], rowsf_v, sem).wait()
    pltpu.sync_copy(rowsf_v, of_ref.at[pl.ds(base, bpw)])
    pltpu.async_copy(side_ref.at[idx_v], rowss_v, sem).wait()
    pltpu.sync_copy(rowss_v, os_ref.at[pl.ds(base, bpw)])


def kernel(features, pos, cam_ids):
    b, n, c = features.shape
    m = n // _SUB
    x2 = jnp.sum(features * features, axis=-1)
    x2col = x2.reshape(b, n, 1)
    x2row = x2.reshape(b, 1, n)

    drow, dcol = pl.pallas_call(
        _knn_kernel,
        grid=(b, n // _BR),
        in_specs=[pl.BlockSpec((1, _BR, c), lambda bb, i: (bb, i, 0)),
                  pl.BlockSpec((1, n, c), lambda bb, i: (bb, 0, 0)),
                  pl.BlockSpec((1, _BR, 1), lambda bb, i: (bb, i, 0)),
                  pl.BlockSpec((1, 1, n), lambda bb, i: (bb, 0, 0))],
        out_specs=[pl.BlockSpec((1, 1, _BR), lambda bb, i: (bb, 0, i)),
                   pl.BlockSpec((1, _BR, 1), lambda bb, i: (bb, i, 0))],
        out_shape=[jax.ShapeDtypeStruct((b, 1, n), jnp.float32),
                   jax.ShapeDtypeStruct((b, n, 1), jnp.float32)],
    )(features, features, x2col, x2row)

    rankrow = pl.pallas_call(
        _rank_kernel,
        grid=(b, n // _BRANK),
        in_specs=[pl.BlockSpec((1, _BRANK, 1), lambda bb, i: (bb, i, 0)),
                  pl.BlockSpec((1, 1, n), lambda bb, i: (bb, 0, 0))],
        out_specs=pl.BlockSpec((1, 1, _BRANK), lambda bb, i: (bb, 0, i)),
        out_shape=jax.ShapeDtypeStruct((b, 1, n), jnp.int32),
    )(dcol, drow)

    inds = pl.pallas_call(
        _inds_kernel,
        grid=(b, m // _BM),
        in_specs=[pl.BlockSpec((1, 1, n), lambda bb, j: (bb, 0, 0))],
        out_specs=pl.BlockSpec((1, _BM, 1), lambda bb, j: (bb, j, 0)),
        out_shape=jax.ShapeDtypeStruct((b, m, 1), jnp.int32),
    )(rankrow)

    feat_tbl = features.reshape(b * n, c)
    side_tbl = jnp.concatenate(
        [pos, cam_ids.astype(jnp.float32)[..., None],
         jnp.zeros((b, n, _DS - 4), jnp.float32)], axis=-1).reshape(b * n, _DS)
    idx_flat = inds.reshape(b * m)

    total = b * m
    nw = 32
    bpw = total // nw
    mesh = plsc.VectorSubcoreMesh(core_axis_name="c", subcore_axis_name="s")
    gath = pl.kernel(
        functools.partial(_sc_gather_body, nw, bpw),
        mesh=mesh,
        out_type=[jax.ShapeDtypeStruct((total, c), jnp.float32),
                  jax.ShapeDtypeStruct((total, _DS), jnp.float32)],
        scratch_types=[pltpu.VMEM((bpw,), jnp.int32),
                       pltpu.VMEM((bpw, c), jnp.float32),
                       pltpu.VMEM((bpw, _DS), jnp.float32),
                       pltpu.SemaphoreType.DMA],
    )
    outf, outs = gath(feat_tbl, side_tbl, idx_flat)

    feats = outf.reshape(b, m, c)
    posg = outs[:, :3].reshape(b, m, 3)
    cam = outs[:, 3].astype(jnp.int32).reshape(b, m)
    return feats, posg, cam


# speedup vs baseline: 1.1669x; 1.0661x over previous
import functools

import jax
import jax.numpy as jnp
from jax import lax
from jax.experimental import pallas as pl
from jax.experimental.pallas import tpu as pltpu
from jax.experimental.pallas import tpu_sc as plsc

_K = 8
_SUB = 4
_BR = 256
_BRANK = 512
_BM = 256
_DS = 128


def _knn_kernel(xr_ref, xa_ref, x2r_ref, x2c_ref, drow_ref, dcol_ref):
    xr = xr_ref[0]
    xa = xa_ref[0]
    n = xa.shape[0]
    dotv = jax.lax.dot_general(xr, xa, (((1,), (1,)), ((), ())),
                               preferred_element_type=jnp.float32)
    x2r = x2r_ref[0]
    x2c = x2c_ref[0]
    d2 = (x2r + x2c) - 2.0 * dotv
    d2 = jnp.maximum(d2, 0.0)
    ms, cnts = [], []
    for _ in range(_K):
        m = jnp.min(d2, axis=1, keepdims=True)
        eq = d2 == m
        cnts.append(jnp.sum(eq.astype(jnp.int32), axis=1, keepdims=True))
        d2 = jnp.where(eq, jnp.inf, d2)
        ms.append(jnp.sqrt(m))
    vals = [None] * _K
    prev = jnp.zeros_like(cnts[0])
    for j in range(_K):
        pj = prev + cnts[j]
        for i in range(_K):
            sel = (prev <= i) & (i < pj)
            v = jnp.where(sel, ms[j], 0.0)
            vals[i] = v if vals[i] is None else vals[i] + v
        prev = pj
    s1 = [vals[i] + vals[i + 4] for i in range(4)]
    s2 = [s1[0] + s1[2], s1[1] + s1[3]]
    dens = (s2[0] + s2[1]) * 0.125
    drow_ref[0, 0, :] = dens[:, 0]
    dcol_ref[0] = dens


def _rank_kernel(dcol_ref, drow_ref, rrow_ref):
    i = pl.program_id(1)
    dn = dcol_ref[0]
    dm = drow_ref[0]
    br, n = dn.shape[0], dm.shape[1]
    cols = jax.lax.broadcasted_iota(jnp.int32, (br, n), 1)
    rows = jax.lax.broadcasted_iota(jnp.int32, (br, n), 0) + i * br
    beats = (dm > dn) | ((dm == dn) & (cols < rows))
    rank = jnp.sum(beats.astype(jnp.int32), axis=1)
    rrow_ref[0, 0, :] = rank


def _inds_kernel(rrow_ref, i_ref):
    bb = pl.program_id(0)
    j = pl.program_id(1)
    rank = rrow_ref[0]
    bm = i_ref.shape[1]
    n = rank.shape[1]
    r = jax.lax.broadcasted_iota(jnp.int32, (bm, n), 0) + j * bm
    eq = rank == r
    cols = jax.lax.broadcasted_iota(jnp.int32, (bm, n), 1)
    inds = jnp.sum(jnp.where(eq, cols, 0), axis=1, keepdims=True)
    i_ref[0] = inds + bb * n


def _sc_gather_body(nw, bpw, feat_ref, side_ref, idx_ref,
                    of_ref, os_ref, idx_v, rowsf_v, rowss_v, sem):
    wid = lax.axis_index("s") * 2 + lax.axis_index("c")
    base = wid * bpw
    pltpu.sync_copy(idx_ref.at[pl.ds(base, bpw)], idx_v)
    pltpu.async_copy(feat_ref.at[idx_v], rowsf_v, sem).wait()
    pltpu.sync_copy(rowsf_v, of_ref.at[pl.ds(base, bpw)])
    pltpu.async_copy(side_ref.at[idx_v], rowss_v, sem).wait()
    pltpu.sync_copy(rowss_v, os_ref.at[pl.ds(base, bpw)])


def kernel(features, pos, cam_ids):
    b, n, c = features.shape
    m = n // _SUB
    x2 = jnp.sum(features * features, axis=-1)
    x2col = x2.reshape(b, n, 1)
    x2row = x2.reshape(b, 1, n)

    drow, dcol = pl.pallas_call(
        _knn_kernel,
        grid=(b, n // _BR),
        in_specs=[pl.BlockSpec((1, _BR, c), lambda bb, i: (bb, i, 0)),
                  pl.BlockSpec((1, n, c), lambda bb, i: (bb, 0, 0)),
                  pl.BlockSpec((1, _BR, 1), lambda bb, i: (bb, i, 0)),
                  pl.BlockSpec((1, 1, n), lambda bb, i: (bb, 0, 0))],
        out_specs=[pl.BlockSpec((1, 1, _BR), lambda bb, i: (bb, 0, i)),
                   pl.BlockSpec((1, _BR, 1), lambda bb, i: (bb, i, 0))],
        out_shape=[jax.ShapeDtypeStruct((b, 1, n), jnp.float32),
                   jax.ShapeDtypeStruct((b, n, 1), jnp.float32)],
    )(features, features, x2col, x2row)

    rankrow = pl.pallas_call(
        _rank_kernel,
        grid=(b, n // _BRANK),
        in_specs=[pl.BlockSpec((1, _BRANK, 1), lambda bb, i: (bb, i, 0)),
                  pl.BlockSpec((1, 1, n), lambda bb, i: (bb, 0, 0))],
        out_specs=pl.BlockSpec((1, 1, _BRANK), lambda bb, i: (bb, 0, i)),
        out_shape=jax.ShapeDtypeStruct((b, 1, n), jnp.int32),
    )(dcol, drow)

    inds = pl.pallas_call(
        _inds_kernel,
        grid=(b, m // _BM),
        in_specs=[pl.BlockSpec((1, 1, n), lambda bb, j: (bb, 0, 0))],
        out_specs=pl.BlockSpec((1, _BM, 1), lambda bb, j: (bb, j, 0)),
        out_shape=jax.ShapeDtypeStruct((b, m, 1), jnp.int32),
    )(rankrow)

    feat_tbl = features.reshape(b * n, c)
    side_tbl = jnp.concatenate(
        [pos, cam_ids.astype(jnp.float32)[..., None],
         jnp.zeros((b, n, _DS - 4), jnp.float32)], axis=-1).reshape(b * n, _DS)
    idx_flat = inds.reshape(b * m)

    total = b * m
    nw = 32
    bpw = total // nw
    mesh = plsc.VectorSubcoreMesh(core_axis_name="c", subcore_axis_name="s")
    gath = pl.kernel(
        functools.partial(_sc_gather_body, nw, bpw),
        mesh=mesh,
        out_type=[jax.ShapeDtypeStruct((total, c), jnp.float32),
                  jax.ShapeDtypeStruct((total, _DS), jnp.float32)],
        scratch_types=[pltpu.VMEM((bpw,), jnp.int32),
                       pltpu.VMEM((bpw, c), jnp.float32),
                       pltpu.VMEM((bpw, _DS), jnp.float32),
                       pltpu.SemaphoreType.DMA],
    )
    outf, outs = gath(feat_tbl, side_tbl, idx_flat)

    feats = outf.reshape(b, m, c)
    posg = outs[:, :3].reshape(b, m, 3)
    cam = outs[:, 3].astype(jnp.int32).reshape(b, m)
    return feats, posg, cam
